# Initial kernel scaffold; baseline (speedup 1.0000x reference)
import jax
import jax.numpy as jnp
from jax.experimental import pallas as pl


def kernel(x, distance_scores, weight, bias):
    raise NotImplementedError("write your pallas kernel here")


# speedup vs baseline: 7.7883x; 7.7883x over previous
import functools

import jax
import jax.numpy as jnp
from jax import lax
from jax.experimental import pallas as pl
from jax.experimental.pallas import tpu as pltpu
from jax.experimental.pallas import tpu_sc as plsc

CAT = 100000
LANES = 128
ROWS = 784
PAD = ROWS * LANES
B = 16384
F = 26
N = B * F
EMB = 32
NW = 32
CHUNK = N // NW
SCL = 16
B4 = N * EMB // LANES
R4 = 2048


def _distances_kernel(s_ref, o_ref):
    s = s_ref[...]
    m = jnp.max(s)
    e = jnp.exp(s - m)
    total = jnp.sum(e)
    lane = lax.broadcasted_iota(jnp.int32, (ROWS, LANES), 1)
    x = e
    for k in (1, 2, 4, 8, 16, 32, 64):
        x = x + jnp.where(lane >= k, pltpu.roll(x, k, axis=1), 0.0)
    r = x[:, LANES - 1:LANES]
    i0 = lax.broadcasted_iota(jnp.int32, (ROWS, ROWS), 0)
    i1 = lax.broadcasted_iota(jnp.int32, (ROWS, ROWS), 1)
    lmat = (i1 < i0).astype(jnp.float32)
    offs = lax.dot_general(lmat, r, (((1,), (0,)), ((), ())),
                           preferred_element_type=jnp.float32,
                           precision=lax.Precision.HIGHEST)
    o_ref[...] = (offs + x - e) / total


def _sc_gather_body(table_hbm, idx_hbm, out_hbm, table_v, idx_v, d_v):
    wid = lax.axis_index("s") * 2 + lax.axis_index("c")
    base = wid * CHUNK
    pltpu.sync_copy(table_hbm, table_v)
    pltpu.sync_copy(idx_hbm.at[pl.ds(base, CHUNK)], idx_v)

    @pl.loop(0, CHUNK, step=8 * SCL)
    def _(i):
        for j in range(8):
            off = i + j * SCL
            iv = idx_v[pl.ds(off, SCL)]
            d_v[pl.ds(off, SCL)] = plsc.load_gather(table_v, [iv])

    pltpu.sync_copy(d_v, out_hbm.at[pl.ds(base, CHUNK)])


@functools.cache
def _sc_gather():
    mesh = plsc.VectorSubcoreMesh(core_axis_name="c", subcore_axis_name="s")
    cp = pltpu.CompilerParams()
    if "needs_layout_passes" in pltpu.CompilerParams.__dataclass_fields__:
        import dataclasses
        cp = dataclasses.replace(cp, needs_layout_passes=False)
    return pl.kernel(
        _sc_gather_body,
        out_type=jax.ShapeDtypeStruct((N,), jnp.float32),
        mesh=mesh,
        compiler_params=cp,
        scratch_types=[
            pltpu.VMEM((PAD,), jnp.float32),
            pltpu.VMEM((CHUNK,), jnp.int32),
            pltpu.VMEM((CHUNK,), jnp.float32),
        ],
    )


def _expand_kernel(d4_ref, mw_ref, bt_ref, o_ref):
    o_ref[...] = lax.dot_general(d4_ref[...], mw_ref[...],
                                 (((1,), (0,)), ((), ())),
                                 preferred_element_type=jnp.float32,
                                 precision=lax.Precision.HIGHEST) + bt_ref[...]


def kernel(x, distance_scores, weight, bias):
    x_flat = x.reshape(-1).astype(jnp.int32)
    s_pad = jnp.pad(distance_scores.astype(jnp.float32),
                    (0, PAD - (CAT - 1)), constant_values=-1e30)
    table2d = pl.pallas_call(
        _distances_kernel,
        out_shape=jax.ShapeDtypeStruct((ROWS, LANES), jnp.float32),
    )(s_pad.reshape(ROWS, LANES))
    d = _sc_gather()(table2d.reshape(PAD), x_flat)
    lane = jnp.arange(LANES)
    onehot = (lane[None, :] // EMB == jnp.arange(4)[:, None]).astype(jnp.float32)
    mw = onehot * jnp.tile(weight.astype(jnp.float32), 4)[None, :]
    bt = jnp.tile(bias.astype(jnp.float32), 4).reshape(1, LANES)
    out2 = pl.pallas_call(
        _expand_kernel,
        grid=(B4 // R4,),
        in_specs=[
            pl.BlockSpec((R4, 4), lambda i: (i, 0)),
            pl.BlockSpec((4, LANES), lambda i: (0, 0)),
            pl.BlockSpec((1, LANES), lambda i: (0, 0)),
        ],
        out_specs=pl.BlockSpec((R4, LANES), lambda i: (i, 0)),
        out_shape=jax.ShapeDtypeStruct((B4, LANES), jnp.float32),
    )(d.reshape(B4, 4), mw, bt)
    return out2.reshape(B, F, EMB)


# speedup vs baseline: 26.0543x; 3.3453x over previous
import functools

import jax
import jax.numpy as jnp
from jax import lax
from jax.experimental import pallas as pl
from jax.experimental.pallas import tpu as pltpu
from jax.experimental.pallas import tpu_sc as plsc

CAT = 100000
LANES = 128
ROWS = 784
PAD = ROWS * LANES
B = 16384
F = 26
N = B * F
EMB = 32
NW = 32
CHUNK = N // NW
SCL = 16
B4 = N * EMB // LANES
R4 = 2048


def _distances_kernel(s_ref, o_ref):
    s = s_ref[...]
    m = jnp.max(s)
    e = jnp.exp(s - m)
    total = jnp.sum(e)
    lane = lax.broadcasted_iota(jnp.int32, (ROWS, LANES), 1)
    x = e
    for k in (1, 2, 4, 8, 16, 32, 64):
        x = x + jnp.where(lane >= k, pltpu.roll(x, k, axis=1), 0.0)
    r = x[:, LANES - 1:LANES]
    i0 = lax.broadcasted_iota(jnp.int32, (ROWS, ROWS), 0)
    i1 = lax.broadcasted_iota(jnp.int32, (ROWS, ROWS), 1)
    lmat = (i1 < i0).astype(jnp.float32)
    offs = lax.dot_general(lmat, r, (((1,), (0,)), ((), ())),
                           preferred_element_type=jnp.float32,
                           precision=lax.Precision.HIGHEST)
    o_ref[...] = (offs + x - e) / total


def _sc_gather_body(table_hbm, idx_hbm, out_hbm, table_v, idx_v, d_v):
    wid = lax.axis_index("s") * 2 + lax.axis_index("c")
    base = wid * CHUNK
    pltpu.sync_copy(table_hbm, table_v)
    pltpu.sync_copy(idx_hbm.at[pl.ds(base, CHUNK)], idx_v)

    @pl.loop(0, CHUNK, step=8 * SCL)
    def _(i):
        for j in range(8):
            off = i + j * SCL
            iv = idx_v[pl.ds(off, SCL)]
            d_v[pl.ds(off, SCL)] = plsc.load_gather(table_v, [iv])

    pltpu.sync_copy(d_v, out_hbm.at[pl.ds(base, CHUNK)])


@functools.cache
def _sc_gather():
    mesh = plsc.VectorSubcoreMesh(core_axis_name="c", subcore_axis_name="s")
    cp = pltpu.CompilerParams()
    if "needs_layout_passes" in pltpu.CompilerParams.__dataclass_fields__:
        import dataclasses
        cp = dataclasses.replace(cp, needs_layout_passes=False)
    return pl.kernel(
        _sc_gather_body,
        out_type=jax.ShapeDtypeStruct((N,), jnp.float32),
        mesh=mesh,
        compiler_params=cp,
        scratch_types=[
            pltpu.VMEM((PAD,), jnp.float32),
            pltpu.VMEM((CHUNK,), jnp.int32),
            pltpu.VMEM((CHUNK,), jnp.float32),
        ],
    )


def _expand_kernel(w_ref, b_ref, d_ref, o_ref):
    dd = d_ref[...]
    for e in range(EMB):
        o_ref[0, e] = w_ref[e] * dd + b_ref[e]


def kernel(x, distance_scores, weight, bias):
    xt_flat = jnp.transpose(x).reshape(-1).astype(jnp.int32)
    s_pad = jnp.pad(distance_scores.astype(jnp.float32),
                    (0, PAD - (CAT - 1)), constant_values=-1e30)
    table2d = pl.pallas_call(
        _distances_kernel,
        out_shape=jax.ShapeDtypeStruct((ROWS, LANES), jnp.float32),
    )(s_pad.reshape(ROWS, LANES))
    dt = _sc_gather()(table2d.reshape(PAD), xt_flat)
    out4 = pl.pallas_call(
        _expand_kernel,
        grid=(F,),
        in_specs=[
            pl.BlockSpec(memory_space=pltpu.SMEM),
            pl.BlockSpec(memory_space=pltpu.SMEM),
            pl.BlockSpec((B // LANES, LANES), lambda f: (f, 0)),
        ],
        out_specs=pl.BlockSpec((1, EMB, B // LANES, LANES),
                               lambda f: (f, 0, 0, 0)),
        out_shape=jax.ShapeDtypeStruct((F, EMB, B // LANES, LANES), jnp.float32),
    )(weight.astype(jnp.float32), bias.astype(jnp.float32),
      dt.reshape(F * B // LANES, LANES))
    return jnp.transpose(out4.reshape(F, EMB, B), (2, 0, 1))
